# baseline (speedup 1.0000x reference)
import functools
import jax
import jax.numpy as jnp
from jax import lax
from jax.experimental import pallas as pl
from jax.experimental.pallas import tpu as pltpu, tpu_sc as plsc

MODEL_DIM = 1024
MAX_LEN = 32768
BATCH = 16384

_info = plsc.get_sparse_core_info()
_NC, _NS = _info.num_cores, _info.num_subcores
_NW = _NC * _NS
_BPW = BATCH // _NW
_CHUNK = 16
_NCHUNK = _BPW // _CHUNK
_NBUF = 7


@functools.partial(
    pl.kernel,
    mesh=plsc.VectorSubcoreMesh(core_axis_name="c", subcore_axis_name="s"),
    out_type=jax.ShapeDtypeStruct((BATCH, MODEL_DIM), jnp.float32),
    scratch_types=(
        [pltpu.VMEM((_BPW,), jnp.int32)]
        + [pltpu.VMEM((_CHUNK, MODEL_DIM), jnp.float32)] * _NBUF
        + [pltpu.SemaphoreType.DMA] * (2 * _NBUF)
    ),
)
def _sc_gather(x_hbm, pe_hbm, out_hbm, idx_v, *bufs_and_sems):
    bufs = bufs_and_sems[:_NBUF]
    in_sems = bufs_and_sems[_NBUF:2 * _NBUF]
    out_sems = bufs_and_sems[2 * _NBUF:]

    wid = lax.axis_index("s") * _NC + lax.axis_index("c")
    base = wid * _BPW
    pltpu.sync_copy(x_hbm.at[pl.ds(base, _BPW)], idx_v)

    def gather(c, slot):
        return pltpu.async_copy(
            pe_hbm.at[idx_v.at[pl.ds(c * _CHUNK, _CHUNK)]],
            bufs[slot], in_sems[slot],
        )

    def put(c, slot):
        return pltpu.async_copy(
            bufs[slot], out_hbm.at[pl.ds(base + c * _CHUNK, _CHUNK)],
            out_sems[slot],
        )

    gathers = [None] * _NBUF
    puts = [None] * _NBUF
    for b in range(_NBUF - 1):
        gathers[b] = gather(b, b)
    for c in range(_NCHUNK):
        slot = c % _NBUF
        pre = c + _NBUF - 1
        if pre < _NCHUNK:
            s2 = pre % _NBUF
            if puts[s2] is not None:
                puts[s2].wait()
            gathers[s2] = gather(pre, s2)
        gathers[slot].wait()
        puts[slot] = put(c, slot)
    for b in range(_NBUF):
        if puts[b] is not None:
            puts[b].wait()


def kernel(x, pe):
    return _sc_gather(x.astype(jnp.int32), pe)
